# Initial kernel scaffold; baseline (speedup 1.0000x reference)
import jax
import jax.numpy as jnp
from jax.experimental import pallas as pl


def kernel(x, type_vec, edge_feas_dict, wg_w1, wg_b1, wg_w2, wg_b2, wg_w3, wg_b3, bg_w1, bg_b1, bg_w2, bg_b2, bg_w3, bg_b3):
    raise NotImplementedError("write your pallas kernel here")


# speedup vs baseline: 3.9618x; 3.9618x over previous
import jax
import jax.numpy as jnp
from jax.experimental import pallas as pl

NT = 8
MEMD = 128
HIDD = 64
IND = 768
OUTD = 768
NTOK = 4096


def _gen_small_kernel(m_ref, ww1_ref, wb1_ref, ww2_ref, wb2_ref,
                      bw1_ref, bb1_ref, bw2_ref, bb2_ref, bw3_ref, bb3_ref,
                      hw_ref, ball_ref):
    m = m_ref[...]
    h = jnp.dot(m, ww1_ref[...], preferred_element_type=jnp.float32) + wb1_ref[...]
    h = jnp.maximum(h, 0.0)
    h = jnp.dot(h, ww2_ref[...], preferred_element_type=jnp.float32) + wb2_ref[...]
    h = jnp.maximum(h, 0.0)
    hw_ref[...] = h
    g = jnp.dot(m, bw1_ref[...], preferred_element_type=jnp.float32) + bb1_ref[...]
    g = jnp.maximum(g, 0.0)
    g = jnp.dot(g, bw2_ref[...], preferred_element_type=jnp.float32) + bb2_ref[...]
    g = jnp.maximum(g, 0.0)
    ball_ref[...] = jnp.dot(g, bw3_ref[...], preferred_element_type=jnp.float32) + bb3_ref[...]


def _wgen_kernel(hw_ref, w3_ref, b3_ref, wout_ref):
    wout_ref[...] = (jnp.dot(hw_ref[...], w3_ref[...],
                             preferred_element_type=jnp.float32)
                     + b3_ref[...])


def _apply_kernel(tv_ref, x_ref, w_ref, b_ref, out_ref):
    xb = x_ref[...]
    tv = tv_ref[...]
    acc = jnp.zeros(out_ref.shape, jnp.float32)
    for t in range(NT):
        mt = tv == t
        xt = jnp.where(mt, xb, 0.0)
        acc = acc + jnp.dot(xt, w_ref[t], preferred_element_type=jnp.float32)
        acc = acc + jnp.where(mt, b_ref[t:t + 1, :], 0.0)
    out_ref[...] = acc


def kernel(x, type_vec, edge_feas_dict,
           wg_w1, wg_b1, wg_w2, wg_b2, wg_w3, wg_b3,
           bg_w1, bg_b1, bg_w2, bg_b2, bg_w3, bg_b3):
    tv = type_vec.astype(jnp.int32).reshape(NTOK, 1)

    hw, ball = pl.pallas_call(
        _gen_small_kernel,
        out_shape=(jax.ShapeDtypeStruct((NT, HIDD), jnp.float32),
                   jax.ShapeDtypeStruct((NT, OUTD), jnp.float32)),
    )(edge_feas_dict,
      wg_w1, wg_b1.reshape(1, HIDD), wg_w2, wg_b2.reshape(1, HIDD),
      bg_w1, bg_b1.reshape(1, HIDD), bg_w2, bg_b2.reshape(1, HIDD),
      bg_w3, bg_b3.reshape(1, OUTD))

    CB = 49152
    ncb = (IND * OUTD) // CB
    w_all = pl.pallas_call(
        _wgen_kernel,
        grid=(ncb,),
        in_specs=[
            pl.BlockSpec((NT, HIDD), lambda j: (0, 0)),
            pl.BlockSpec((HIDD, CB), lambda j: (0, j)),
            pl.BlockSpec((1, CB), lambda j: (0, j)),
        ],
        out_specs=pl.BlockSpec((NT, CB), lambda j: (0, j)),
        out_shape=jax.ShapeDtypeStruct((NT, IND * OUTD), jnp.float32),
    )(hw, wg_w3, wg_b3.reshape(1, IND * OUTD))
    w_all = w_all.reshape(NT, IND, OUTD)

    BN = 1024
    nnb = NTOK // BN
    out = pl.pallas_call(
        _apply_kernel,
        grid=(nnb,),
        in_specs=[
            pl.BlockSpec((BN, 1), lambda n: (n, 0)),
            pl.BlockSpec((BN, IND), lambda n: (n, 0)),
            pl.BlockSpec((NT, IND, OUTD), lambda n: (0, 0, 0)),
            pl.BlockSpec((NT, OUTD), lambda n: (0, 0)),
        ],
        out_specs=pl.BlockSpec((BN, OUTD), lambda n: (n, 0)),
        out_shape=jax.ShapeDtypeStruct((NTOK, OUTD), jnp.float32),
    )(tv, x, w_all, ball)
    return out


# speedup vs baseline: 4.9069x; 1.2386x over previous
import jax
import jax.numpy as jnp
from jax.experimental import pallas as pl

NT = 8
MEMD = 128
HIDD = 64
IND = 768
OUTD = 768
NTOK = 4096


def _gen_small_kernel(m_ref, ww1_ref, wb1_ref, ww2_ref, wb2_ref,
                      bw1_ref, bb1_ref, bw2_ref, bb2_ref, bw3_ref, bb3_ref,
                      hw_ref, ball_ref):
    m = m_ref[...]
    h = jnp.dot(m, ww1_ref[...], preferred_element_type=jnp.float32) + wb1_ref[...]
    h = jnp.maximum(h, 0.0)
    h = jnp.dot(h, ww2_ref[...], preferred_element_type=jnp.float32) + wb2_ref[...]
    h = jnp.maximum(h, 0.0)
    hw_ref[...] = h
    g = jnp.dot(m, bw1_ref[...], preferred_element_type=jnp.float32) + bb1_ref[...]
    g = jnp.maximum(g, 0.0)
    g = jnp.dot(g, bw2_ref[...], preferred_element_type=jnp.float32) + bb2_ref[...]
    g = jnp.maximum(g, 0.0)
    ball_ref[...] = jnp.dot(g, bw3_ref[...], preferred_element_type=jnp.float32) + bb3_ref[...]


def _wgen_kernel(hw_ref, w3_ref, b3_ref, wout_ref):
    w2 = (jnp.dot(hw_ref[...], w3_ref[...],
                  preferred_element_type=jnp.float32)
          + b3_ref[...])
    wout_ref[...] = w2.reshape(wout_ref.shape)


def _apply_kernel(tv_ref, x_ref, w_ref, b_ref, out_ref):
    xb = x_ref[...]
    tv = tv_ref[...]
    acc = jnp.zeros(out_ref.shape, jnp.float32)
    for t in range(NT):
        mt = tv == t
        xt = jnp.where(mt, xb, 0.0)
        acc = acc + jnp.dot(xt, w_ref[t], preferred_element_type=jnp.float32)
        acc = acc + jnp.where(mt, b_ref[t:t + 1, :], 0.0)
    out_ref[...] = acc


def kernel(x, type_vec, edge_feas_dict,
           wg_w1, wg_b1, wg_w2, wg_b2, wg_w3, wg_b3,
           bg_w1, bg_b1, bg_w2, bg_b2, bg_w3, bg_b3):
    tv = type_vec.astype(jnp.int32).reshape(NTOK, 1)

    hw, ball = pl.pallas_call(
        _gen_small_kernel,
        out_shape=(jax.ShapeDtypeStruct((NT, HIDD), jnp.float32),
                   jax.ShapeDtypeStruct((NT, OUTD), jnp.float32)),
    )(edge_feas_dict,
      wg_w1, wg_b1.reshape(1, HIDD), wg_w2, wg_b2.reshape(1, HIDD),
      bg_w1, bg_b1.reshape(1, HIDD), bg_w2, bg_b2.reshape(1, HIDD),
      bg_w3, bg_b3.reshape(1, OUTD))

    CB = 49152
    ncb = (IND * OUTD) // CB
    w_all = pl.pallas_call(
        _wgen_kernel,
        grid=(ncb,),
        in_specs=[
            pl.BlockSpec((NT, HIDD), lambda j: (0, 0)),
            pl.BlockSpec((HIDD, CB), lambda j: (0, j)),
            pl.BlockSpec((1, CB), lambda j: (0, j)),
        ],
        out_specs=pl.BlockSpec((NT, CB // OUTD, OUTD), lambda j: (0, j, 0)),
        out_shape=jax.ShapeDtypeStruct((NT, IND, OUTD), jnp.float32),
    )(hw, wg_w3, wg_b3.reshape(1, IND * OUTD))

    BN = 1024
    nnb = NTOK // BN
    out = pl.pallas_call(
        _apply_kernel,
        grid=(nnb,),
        in_specs=[
            pl.BlockSpec((BN, 1), lambda n: (n, 0)),
            pl.BlockSpec((BN, IND), lambda n: (n, 0)),
            pl.BlockSpec((NT, IND, OUTD), lambda n: (0, 0, 0)),
            pl.BlockSpec((NT, OUTD), lambda n: (0, 0)),
        ],
        out_specs=pl.BlockSpec((BN, OUTD), lambda n: (n, 0)),
        out_shape=jax.ShapeDtypeStruct((NTOK, OUTD), jnp.float32),
    )(tv, x, w_all, ball)
    return out


# speedup vs baseline: 5.1958x; 1.0589x over previous
import jax
import jax.numpy as jnp
from jax.experimental import pallas as pl

NT = 8
MEMD = 128
HIDD = 64
IND = 768
OUTD = 768
NTOK = 4096


def _gen_small_kernel(m_ref, ww1_ref, wb1_ref, ww2_ref, wb2_ref,
                      bw1_ref, bb1_ref, bw2_ref, bb2_ref, bw3_ref, bb3_ref,
                      hw_ref, ball_ref):
    m = m_ref[...]
    h = jnp.dot(m, ww1_ref[...], preferred_element_type=jnp.float32) + wb1_ref[...]
    h = jnp.maximum(h, 0.0)
    h = jnp.dot(h, ww2_ref[...], preferred_element_type=jnp.float32) + wb2_ref[...]
    h = jnp.maximum(h, 0.0)
    hw_ref[...] = h
    g = jnp.dot(m, bw1_ref[...], preferred_element_type=jnp.float32) + bb1_ref[...]
    g = jnp.maximum(g, 0.0)
    g = jnp.dot(g, bw2_ref[...], preferred_element_type=jnp.float32) + bb2_ref[...]
    g = jnp.maximum(g, 0.0)
    ball_ref[...] = jnp.dot(g, bw3_ref[...], preferred_element_type=jnp.float32) + bb3_ref[...]


def _wgen_kernel(hw_ref, w3_ref, b3_ref, wout_ref):
    w2 = (jnp.dot(hw_ref[...], w3_ref[...],
                  preferred_element_type=jnp.float32)
          + b3_ref[...])
    wout_ref[...] = w2.reshape(wout_ref.shape).astype(jnp.bfloat16)


def _apply_kernel(tv_ref, x_ref, w_ref, b_ref, out_ref):
    xb = x_ref[...].astype(jnp.bfloat16)
    tv = tv_ref[...]
    acc = jnp.zeros(out_ref.shape, jnp.float32)
    for t in range(NT):
        mt = tv == t
        xt = jnp.where(mt, xb, jnp.bfloat16(0.0))
        acc = acc + jnp.dot(xt, w_ref[t], preferred_element_type=jnp.float32)
        acc = acc + jnp.where(mt, b_ref[t:t + 1, :], 0.0)
    out_ref[...] = acc


def kernel(x, type_vec, edge_feas_dict,
           wg_w1, wg_b1, wg_w2, wg_b2, wg_w3, wg_b3,
           bg_w1, bg_b1, bg_w2, bg_b2, bg_w3, bg_b3):
    tv = type_vec.astype(jnp.int32).reshape(NTOK, 1)

    hw, ball = pl.pallas_call(
        _gen_small_kernel,
        out_shape=(jax.ShapeDtypeStruct((NT, HIDD), jnp.float32),
                   jax.ShapeDtypeStruct((NT, OUTD), jnp.float32)),
    )(edge_feas_dict,
      wg_w1, wg_b1.reshape(1, HIDD), wg_w2, wg_b2.reshape(1, HIDD),
      bg_w1, bg_b1.reshape(1, HIDD), bg_w2, bg_b2.reshape(1, HIDD),
      bg_w3, bg_b3.reshape(1, OUTD))

    CB = 49152
    ncb = (IND * OUTD) // CB
    w_all = pl.pallas_call(
        _wgen_kernel,
        grid=(ncb,),
        in_specs=[
            pl.BlockSpec((NT, HIDD), lambda j: (0, 0)),
            pl.BlockSpec((HIDD, CB), lambda j: (0, j)),
            pl.BlockSpec((1, CB), lambda j: (0, j)),
        ],
        out_specs=pl.BlockSpec((NT, CB // OUTD, OUTD), lambda j: (0, j, 0)),
        out_shape=jax.ShapeDtypeStruct((NT, IND, OUTD), jnp.bfloat16),
    )(hw, wg_w3, wg_b3.reshape(1, IND * OUTD))

    BN = 1024
    nnb = NTOK // BN
    out = pl.pallas_call(
        _apply_kernel,
        grid=(nnb,),
        in_specs=[
            pl.BlockSpec((BN, 1), lambda n: (n, 0)),
            pl.BlockSpec((BN, IND), lambda n: (n, 0)),
            pl.BlockSpec((NT, IND, OUTD), lambda n: (0, 0, 0)),
            pl.BlockSpec((NT, OUTD), lambda n: (0, 0)),
        ],
        out_specs=pl.BlockSpec((BN, OUTD), lambda n: (n, 0)),
        out_shape=jax.ShapeDtypeStruct((NTOK, OUTD), jnp.float32),
    )(tv, x, w_all, ball)
    return out
